# baseline (speedup 1.0000x reference)
import jax
import jax.numpy as jnp
from jax import lax
from jax.experimental import pallas as pl
from jax.experimental.pallas import tpu as pltpu
from jax.experimental.pallas import tpu_sc as plsc

N = 10000
NP = 10240
NPH = NP // 2
D = 128
DH = D // 2
E = 320000
NC = 2
NS = 16
CHUNK = 96
SCH = 4
NSB = 54
EPT = CHUNK * SCH * NSB
E_PAD = NS * EPT
RPH = NPH // NS

_LANES = 16


def _sc_body(x_hbm, col_hbm, row_hbm, a0_hbm, a1_hbm, a2_hbm, a3_hbm,
             zero_hbm, out_hbm, colv, rowv, a0v, a1v, a2v, a3v,
             buf_a, buf_b, xs, acc, ga, gb, sa, sb_sem):
    c = lax.axis_index("c")
    s = lax.axis_index("s")

    pltpu.sync_copy(x_hbm.at[c, pl.ds(s * RPH, RPH)],
                    xs.at[pl.ds(s * RPH, RPH)])
    pltpu.sync_copy(zero_hbm, acc.at[pl.ds(s * RPH, RPH)])
    plsc.subcore_barrier()

    def gather_start(k, buf, sem):
        pltpu.async_copy(xs.at[colv.at[k]], buf, sem)

    def gather_wait(buf, sem):
        pltpu.make_async_copy(xs.at[colv.at[0]], buf, sem).wait()

    def scat_start(k, buf, sem):
        pltpu.async_copy(buf, acc.at[rowv.at[k]], sem, add=True)

    def scat_wait(buf, sem):
        pltpu.make_async_copy(buf, acc.at[rowv.at[0]], sem).wait()

    def scale_chunk(k, rows):
        base = k * CHUNK
        for g in range(CHUNK // _LANES):
            o = base + g * _LANES
            a0g = a0v[pl.ds(o, _LANES)]
            a1g = a1v[pl.ds(o, _LANES)]
            a2g = a2v[pl.ds(o, _LANES)]
            a3g = a3v[pl.ds(o, _LANES)]
            for j in range(_LANES):
                e = g * _LANES + j
                s0 = jnp.full((_LANES,), a0g[j], jnp.float32)
                s1 = jnp.full((_LANES,), a1g[j], jnp.float32)
                s2 = jnp.full((_LANES,), a2g[j], jnp.float32)
                s3 = jnp.full((_LANES,), a3g[j], jnp.float32)
                for v in range(DH // _LANES):
                    sl_lo = pl.ds(v * _LANES, _LANES)
                    sl_hi = pl.ds(DH + v * _LANES, _LANES)
                    lo = rows[e, sl_lo]
                    hi = rows[e, sl_hi]
                    rows[e, sl_lo] = lo * s0 + hi * s1
                    rows[e, sl_hi] = lo * s2 + hi * s3

    def sb_body(sb, carry):
        pltpu.sync_copy(col_hbm.at[s, sb], colv)
        pltpu.sync_copy(row_hbm.at[s, sb], rowv)
        pltpu.sync_copy(a0_hbm.at[s, sb], a0v)
        pltpu.sync_copy(a1_hbm.at[s, sb], a1v)
        pltpu.sync_copy(a2_hbm.at[s, sb], a2v)
        pltpu.sync_copy(a3_hbm.at[s, sb], a3v)

        @pl.when(sb > 0)
        def _():
            scat_wait(buf_a, sa)

        gather_start(0, buf_a, ga)

        def pair_body(j, carry2):
            k0 = 2 * j
            k1 = k0 + 1
            gather_wait(buf_a, ga)
            scale_chunk(k0, buf_a)
            scat_start(k0, buf_a, sa)

            not_first = jnp.logical_or(sb > 0, j > 0)

            @pl.when(not_first)
            def _():
                scat_wait(buf_b, sb_sem)
                gather_start(k1, buf_b, gb)

            @pl.when(jnp.logical_not(not_first))
            def _():
                gather_start(k1, buf_b, gb)

            gather_wait(buf_b, gb)
            scale_chunk(k1, buf_b)
            scat_start(k1, buf_b, sb_sem)

            @pl.when(k0 + 2 < SCH)
            def _():
                scat_wait(buf_a, sa)
                gather_start(k0 + 2, buf_a, ga)

            return carry2

        lax.fori_loop(0, SCH // 2, pair_body, 0)
        return carry

    lax.fori_loop(0, NSB, sb_body, 0)
    scat_wait(buf_a, sa)
    scat_wait(buf_b, sb_sem)
    plsc.subcore_barrier()
    pltpu.sync_copy(acc.at[pl.ds(s * RPH, RPH)],
                    out_hbm.at[c, pl.ds(s * RPH, RPH)])


_sc_aggregate = pl.kernel(
    _sc_body,
    out_type=jax.ShapeDtypeStruct((NC, NPH, D), jnp.float32),
    mesh=plsc.VectorSubcoreMesh(
        core_axis_name="c", subcore_axis_name="s",
        num_cores=NC, num_subcores=NS),
    scratch_types=[
        pltpu.VMEM((SCH, CHUNK), jnp.int32),
        pltpu.VMEM((SCH, CHUNK), jnp.int32),
        pltpu.VMEM((SCH * CHUNK,), jnp.float32),
        pltpu.VMEM((SCH * CHUNK,), jnp.float32),
        pltpu.VMEM((SCH * CHUNK,), jnp.float32),
        pltpu.VMEM((SCH * CHUNK,), jnp.float32),
        pltpu.VMEM((CHUNK, D), jnp.float32),
        pltpu.VMEM((CHUNK, D), jnp.float32),
        pltpu.VMEM_SHARED((NPH, D), jnp.float32),
        pltpu.VMEM_SHARED((NPH, D), jnp.float32),
        pltpu.SemaphoreType.DMA,
        pltpu.SemaphoreType.DMA,
        pltpu.SemaphoreType.DMA,
        pltpu.SemaphoreType.DMA,
    ],
)

_BM = 1024


def _tc_body(p_ref, w_ref, o_ref):
    o_ref[...] = (
        jnp.dot(p_ref[0], w_ref[pl.ds(0, DH), :],
                preferred_element_type=jnp.float32)
        + jnp.dot(p_ref[1], w_ref[pl.ds(DH, DH), :],
                  preferred_element_type=jnp.float32))


def _tc_matmul(partials, weight):
    return pl.pallas_call(
        _tc_body,
        grid=(NP // _BM,),
        in_specs=[
            pl.BlockSpec((NC, _BM, DH), lambda i: (0, i, 0)),
            pl.BlockSpec((D, D), lambda i: (0, 0)),
        ],
        out_specs=pl.BlockSpec((_BM, D), lambda i: (i, 0)),
        out_shape=jax.ShapeDtypeStruct((NP, D), jnp.float32),
    )(partials, weight)


@jax.jit
def _impl(x, edge_index, adj_values, weight):
    row = edge_index[0]
    col = edge_index[1]
    col2 = jnp.zeros((E_PAD,), jnp.int32).at[:E].set(col >> 1)
    row2 = jnp.zeros((E_PAD,), jnp.int32).at[:E].set(row >> 1)
    cp = (col & 1).astype(jnp.float32)
    rp = (row & 1).astype(jnp.float32)
    a0 = jnp.zeros((E_PAD,), jnp.float32).at[:E].set(
        adj_values * (1 - cp) * (1 - rp))
    a1 = jnp.zeros((E_PAD,), jnp.float32).at[:E].set(
        adj_values * cp * (1 - rp))
    a2 = jnp.zeros((E_PAD,), jnp.float32).at[:E].set(
        adj_values * (1 - cp) * rp)
    a3 = jnp.zeros((E_PAD,), jnp.float32).at[:E].set(
        adj_values * cp * rp)
    col2 = col2.reshape(NS, NSB, SCH, CHUNK)
    row2 = row2.reshape(NS, NSB, SCH, CHUNK)
    a0 = a0.reshape(NS, NSB, SCH * CHUNK)
    a1 = a1.reshape(NS, NSB, SCH * CHUNK)
    a2 = a2.reshape(NS, NSB, SCH * CHUNK)
    a3 = a3.reshape(NS, NSB, SCH * CHUNK)
    xh = x.reshape(N, NC, DH).transpose(1, 0, 2)
    xp = jnp.zeros((NC, NP, DH), jnp.float32).at[:, :N].set(xh)
    x_pack = xp.reshape(NC, NPH, D)
    zeros = jnp.zeros((RPH, D), jnp.float32)
    partials = _sc_aggregate(x_pack, col2, row2, a0, a1, a2, a3, zeros)
    pr = partials.reshape(NC, NP, DH)
    return _tc_matmul(pr, weight)[:N]


def kernel(x, edge_index, adj_values, weight):
    return _impl(x, edge_index, adj_values, weight)


# speedup vs baseline: 1.4555x; 1.4555x over previous
import jax
import jax.numpy as jnp
from jax import lax
from jax.experimental import pallas as pl
from jax.experimental.pallas import tpu as pltpu
from jax.experimental.pallas import tpu_sc as plsc

N = 10000
NP = 10240
NPH = NP // 2
D = 128
DH = D // 2
E = 320000
NC = 2
NS = 16
CHUNK = 64
SCH = 16
NSB = 20
EPT = CHUNK * SCH * NSB
E_PAD = NS * EPT
RPH = NPH // NS

_LANES = 16


def _sc_body(x_hbm, col_hbm, row_hbm, a0_hbm, a1_hbm, a2_hbm, a3_hbm,
             zero_hbm, out_hbm, colv, rowv, a0v, a1v, a2v, a3v,
             buf_a, buf_b, xs, acc, ga, gb, sa, sb_sem):
    c = lax.axis_index("c")
    s = lax.axis_index("s")

    pltpu.sync_copy(x_hbm.at[c, pl.ds(s * RPH, RPH)],
                    xs.at[pl.ds(s * RPH, RPH)])
    pltpu.sync_copy(zero_hbm, acc.at[pl.ds(s * RPH, RPH)])
    plsc.subcore_barrier()

    def gather_start(k, buf, sem):
        pltpu.async_copy(xs.at[colv.at[k]], buf, sem)

    def gather_wait(buf, sem):
        pltpu.make_async_copy(xs.at[colv.at[0]], buf, sem).wait()

    def scat_start(k, buf, sem):
        pltpu.async_copy(buf, acc.at[rowv.at[k]], sem, add=True)

    def scat_wait(buf, sem):
        pltpu.make_async_copy(buf, acc.at[rowv.at[0]], sem).wait()

    def scale_chunk(k, rows):
        base = k * CHUNK
        for g in range(CHUNK // _LANES):
            o = base + g * _LANES
            a0g = a0v[pl.ds(o, _LANES)]
            a1g = a1v[pl.ds(o, _LANES)]
            a2g = a2v[pl.ds(o, _LANES)]
            a3g = a3v[pl.ds(o, _LANES)]
            for j in range(_LANES):
                e = g * _LANES + j
                s0 = jnp.full((_LANES,), a0g[j], jnp.float32)
                s1 = jnp.full((_LANES,), a1g[j], jnp.float32)
                s2 = jnp.full((_LANES,), a2g[j], jnp.float32)
                s3 = jnp.full((_LANES,), a3g[j], jnp.float32)
                for v in range(DH // _LANES):
                    sl_lo = pl.ds(v * _LANES, _LANES)
                    sl_hi = pl.ds(DH + v * _LANES, _LANES)
                    lo = rows[e, sl_lo]
                    hi = rows[e, sl_hi]
                    rows[e, sl_lo] = lo * s0 + hi * s1
                    rows[e, sl_hi] = lo * s2 + hi * s3

    def sb_body(sb, carry):
        pltpu.sync_copy(col_hbm.at[s, sb], colv)
        pltpu.sync_copy(row_hbm.at[s, sb], rowv)
        pltpu.sync_copy(a0_hbm.at[s, sb], a0v)
        pltpu.sync_copy(a1_hbm.at[s, sb], a1v)
        pltpu.sync_copy(a2_hbm.at[s, sb], a2v)
        pltpu.sync_copy(a3_hbm.at[s, sb], a3v)

        @pl.when(sb > 0)
        def _():
            scat_wait(buf_a, sa)

        gather_start(0, buf_a, ga)

        def pair_body(j, carry2):
            k0 = 2 * j
            k1 = k0 + 1
            gather_wait(buf_a, ga)
            scale_chunk(k0, buf_a)
            scat_start(k0, buf_a, sa)

            not_first = jnp.logical_or(sb > 0, j > 0)

            @pl.when(not_first)
            def _():
                scat_wait(buf_b, sb_sem)
                gather_start(k1, buf_b, gb)

            @pl.when(jnp.logical_not(not_first))
            def _():
                gather_start(k1, buf_b, gb)

            gather_wait(buf_b, gb)
            scale_chunk(k1, buf_b)
            scat_start(k1, buf_b, sb_sem)

            @pl.when(k0 + 2 < SCH)
            def _():
                scat_wait(buf_a, sa)
                gather_start(k0 + 2, buf_a, ga)

            return carry2

        lax.fori_loop(0, SCH // 2, pair_body, 0)
        return carry

    lax.fori_loop(0, NSB, sb_body, 0)
    scat_wait(buf_a, sa)
    scat_wait(buf_b, sb_sem)
    plsc.subcore_barrier()
    pltpu.sync_copy(acc.at[pl.ds(s * RPH, RPH)],
                    out_hbm.at[c, pl.ds(s * RPH, RPH)])


_sc_aggregate = pl.kernel(
    _sc_body,
    out_type=jax.ShapeDtypeStruct((NC, NPH, D), jnp.float32),
    mesh=plsc.VectorSubcoreMesh(
        core_axis_name="c", subcore_axis_name="s",
        num_cores=NC, num_subcores=NS),
    scratch_types=[
        pltpu.VMEM((SCH, CHUNK), jnp.int32),
        pltpu.VMEM((SCH, CHUNK), jnp.int32),
        pltpu.VMEM((SCH * CHUNK,), jnp.float32),
        pltpu.VMEM((SCH * CHUNK,), jnp.float32),
        pltpu.VMEM((SCH * CHUNK,), jnp.float32),
        pltpu.VMEM((SCH * CHUNK,), jnp.float32),
        pltpu.VMEM((CHUNK, D), jnp.float32),
        pltpu.VMEM((CHUNK, D), jnp.float32),
        pltpu.VMEM_SHARED((NPH, D), jnp.float32),
        pltpu.VMEM_SHARED((NPH, D), jnp.float32),
        pltpu.SemaphoreType.DMA,
        pltpu.SemaphoreType.DMA,
        pltpu.SemaphoreType.DMA,
        pltpu.SemaphoreType.DMA,
    ],
)

_BM = 1024


def _tc_body(p_ref, w_ref, o_ref):
    o_ref[...] = (
        jnp.dot(p_ref[0], w_ref[pl.ds(0, DH), :],
                preferred_element_type=jnp.float32)
        + jnp.dot(p_ref[1], w_ref[pl.ds(DH, DH), :],
                  preferred_element_type=jnp.float32))


def _tc_matmul(partials, weight):
    return pl.pallas_call(
        _tc_body,
        grid=(NP // _BM,),
        in_specs=[
            pl.BlockSpec((NC, _BM, DH), lambda i: (0, i, 0)),
            pl.BlockSpec((D, D), lambda i: (0, 0)),
        ],
        out_specs=pl.BlockSpec((_BM, D), lambda i: (i, 0)),
        out_shape=jax.ShapeDtypeStruct((NP, D), jnp.float32),
    )(partials, weight)


@jax.jit
def _impl(x, edge_index, adj_values, weight):
    row = edge_index[0]
    col = edge_index[1]
    col2 = jnp.zeros((E_PAD,), jnp.int32).at[:E].set(col >> 1)
    row2 = jnp.zeros((E_PAD,), jnp.int32).at[:E].set(row >> 1)
    cp = (col & 1).astype(jnp.float32)
    rp = (row & 1).astype(jnp.float32)
    a0 = jnp.zeros((E_PAD,), jnp.float32).at[:E].set(
        adj_values * (1 - cp) * (1 - rp))
    a1 = jnp.zeros((E_PAD,), jnp.float32).at[:E].set(
        adj_values * cp * (1 - rp))
    a2 = jnp.zeros((E_PAD,), jnp.float32).at[:E].set(
        adj_values * (1 - cp) * rp)
    a3 = jnp.zeros((E_PAD,), jnp.float32).at[:E].set(
        adj_values * cp * rp)
    col2 = col2.reshape(NS, NSB, SCH, CHUNK)
    row2 = row2.reshape(NS, NSB, SCH, CHUNK)
    a0 = a0.reshape(NS, NSB, SCH * CHUNK)
    a1 = a1.reshape(NS, NSB, SCH * CHUNK)
    a2 = a2.reshape(NS, NSB, SCH * CHUNK)
    a3 = a3.reshape(NS, NSB, SCH * CHUNK)
    xh = x.reshape(N, NC, DH).transpose(1, 0, 2)
    xp = jnp.zeros((NC, NP, DH), jnp.float32).at[:, :N].set(xh)
    x_pack = xp.reshape(NC, NPH, D)
    zeros = jnp.zeros((RPH, D), jnp.float32)
    partials = _sc_aggregate(x_pack, col2, row2, a0, a1, a2, a3, zeros)
    pr = partials.reshape(NC, NP, DH)
    return _tc_matmul(pr, weight)[:N]


def kernel(x, edge_index, adj_values, weight):
    return _impl(x, edge_index, adj_values, weight)


# speedup vs baseline: 1.4806x; 1.0172x over previous
import jax
import jax.numpy as jnp
from jax import lax
from jax.experimental import pallas as pl
from jax.experimental.pallas import tpu as pltpu
from jax.experimental.pallas import tpu_sc as plsc

N = 10000
NP = 10240
NPH = NP // 2
D = 128
DH = D // 2
E = 320000
NC = 2
NS = 16
CHUNK = 64
SCH = 20
NSB = 16
EPT = CHUNK * SCH * NSB
E_PAD = NS * EPT
RPH = NPH // NS

_LANES = 16


def _sc_body(x_hbm, col_hbm, row_hbm, a0_hbm, a1_hbm, a2_hbm, a3_hbm,
             zero_hbm, out_hbm, colv, rowv, a0v, a1v, a2v, a3v,
             buf_a, buf_b, xs, acc, ga, gb, sa, sb_sem):
    c = lax.axis_index("c")
    s = lax.axis_index("s")

    pltpu.sync_copy(x_hbm.at[c, pl.ds(s * RPH, RPH)],
                    xs.at[pl.ds(s * RPH, RPH)])
    pltpu.sync_copy(zero_hbm, acc.at[pl.ds(s * RPH, RPH)])
    plsc.subcore_barrier()

    def gather_start(k, buf, sem):
        pltpu.async_copy(xs.at[colv.at[k]], buf, sem)

    def gather_wait(buf, sem):
        pltpu.make_async_copy(xs.at[colv.at[0]], buf, sem).wait()

    def scat_start(k, buf, sem):
        pltpu.async_copy(buf, acc.at[rowv.at[k]], sem, add=True)

    def scat_wait(buf, sem):
        pltpu.make_async_copy(buf, acc.at[rowv.at[0]], sem).wait()

    def scale_chunk(k, rows):
        base = k * CHUNK
        for g in range(CHUNK // _LANES):
            o = base + g * _LANES
            a0g = a0v[pl.ds(o, _LANES)]
            a1g = a1v[pl.ds(o, _LANES)]
            a2g = a2v[pl.ds(o, _LANES)]
            a3g = a3v[pl.ds(o, _LANES)]
            for j in range(_LANES):
                e = g * _LANES + j
                s0 = jnp.full((_LANES,), a0g[j], jnp.float32)
                s1 = jnp.full((_LANES,), a1g[j], jnp.float32)
                s2 = jnp.full((_LANES,), a2g[j], jnp.float32)
                s3 = jnp.full((_LANES,), a3g[j], jnp.float32)
                for v in range(DH // _LANES):
                    sl_lo = pl.ds(v * _LANES, _LANES)
                    sl_hi = pl.ds(DH + v * _LANES, _LANES)
                    lo = rows[e, sl_lo]
                    hi = rows[e, sl_hi]
                    rows[e, sl_lo] = lo * s0 + hi * s1
                    rows[e, sl_hi] = lo * s2 + hi * s3

    def sb_body(sb, carry):
        pltpu.sync_copy(col_hbm.at[s, sb], colv)
        pltpu.sync_copy(row_hbm.at[s, sb], rowv)
        pltpu.sync_copy(a0_hbm.at[s, sb], a0v)
        pltpu.sync_copy(a1_hbm.at[s, sb], a1v)
        pltpu.sync_copy(a2_hbm.at[s, sb], a2v)
        pltpu.sync_copy(a3_hbm.at[s, sb], a3v)

        @pl.when(sb > 0)
        def _():
            scat_wait(buf_a, sa)

        gather_start(0, buf_a, ga)

        def pair_body(j, carry2):
            k0 = 2 * j
            k1 = k0 + 1
            gather_wait(buf_a, ga)
            scale_chunk(k0, buf_a)
            scat_start(k0, buf_a, sa)

            not_first = jnp.logical_or(sb > 0, j > 0)

            @pl.when(not_first)
            def _():
                scat_wait(buf_b, sb_sem)
                gather_start(k1, buf_b, gb)

            @pl.when(jnp.logical_not(not_first))
            def _():
                gather_start(k1, buf_b, gb)

            gather_wait(buf_b, gb)
            scale_chunk(k1, buf_b)
            scat_start(k1, buf_b, sb_sem)

            @pl.when(k0 + 2 < SCH)
            def _():
                scat_wait(buf_a, sa)
                gather_start(k0 + 2, buf_a, ga)

            return carry2

        lax.fori_loop(0, SCH // 2, pair_body, 0)
        return carry

    lax.fori_loop(0, NSB, sb_body, 0)
    scat_wait(buf_a, sa)
    scat_wait(buf_b, sb_sem)
    plsc.subcore_barrier()
    pltpu.sync_copy(acc.at[pl.ds(s * RPH, RPH)],
                    out_hbm.at[c, pl.ds(s * RPH, RPH)])


_sc_aggregate = pl.kernel(
    _sc_body,
    out_type=jax.ShapeDtypeStruct((NC, NPH, D), jnp.float32),
    mesh=plsc.VectorSubcoreMesh(
        core_axis_name="c", subcore_axis_name="s",
        num_cores=NC, num_subcores=NS),
    scratch_types=[
        pltpu.VMEM((SCH, CHUNK), jnp.int32),
        pltpu.VMEM((SCH, CHUNK), jnp.int32),
        pltpu.VMEM((SCH * CHUNK,), jnp.float32),
        pltpu.VMEM((SCH * CHUNK,), jnp.float32),
        pltpu.VMEM((SCH * CHUNK,), jnp.float32),
        pltpu.VMEM((SCH * CHUNK,), jnp.float32),
        pltpu.VMEM((CHUNK, D), jnp.float32),
        pltpu.VMEM((CHUNK, D), jnp.float32),
        pltpu.VMEM_SHARED((NPH, D), jnp.float32),
        pltpu.VMEM_SHARED((NPH, D), jnp.float32),
        pltpu.SemaphoreType.DMA,
        pltpu.SemaphoreType.DMA,
        pltpu.SemaphoreType.DMA,
        pltpu.SemaphoreType.DMA,
    ],
)

_BM = 1024


def _tc_body(p_ref, w_ref, o_ref):
    o_ref[...] = (
        jnp.dot(p_ref[0], w_ref[pl.ds(0, DH), :],
                preferred_element_type=jnp.float32)
        + jnp.dot(p_ref[1], w_ref[pl.ds(DH, DH), :],
                  preferred_element_type=jnp.float32))


def _tc_matmul(partials, weight):
    return pl.pallas_call(
        _tc_body,
        grid=(NP // _BM,),
        in_specs=[
            pl.BlockSpec((NC, _BM, DH), lambda i: (0, i, 0)),
            pl.BlockSpec((D, D), lambda i: (0, 0)),
        ],
        out_specs=pl.BlockSpec((_BM, D), lambda i: (i, 0)),
        out_shape=jax.ShapeDtypeStruct((NP, D), jnp.float32),
    )(partials, weight)


@jax.jit
def _impl(x, edge_index, adj_values, weight):
    row = edge_index[0]
    col = edge_index[1]
    col2 = jnp.zeros((E_PAD,), jnp.int32).at[:E].set(col >> 1)
    row2 = jnp.zeros((E_PAD,), jnp.int32).at[:E].set(row >> 1)
    cp = (col & 1).astype(jnp.float32)
    rp = (row & 1).astype(jnp.float32)
    a0 = jnp.zeros((E_PAD,), jnp.float32).at[:E].set(
        adj_values * (1 - cp) * (1 - rp))
    a1 = jnp.zeros((E_PAD,), jnp.float32).at[:E].set(
        adj_values * cp * (1 - rp))
    a2 = jnp.zeros((E_PAD,), jnp.float32).at[:E].set(
        adj_values * (1 - cp) * rp)
    a3 = jnp.zeros((E_PAD,), jnp.float32).at[:E].set(
        adj_values * cp * rp)
    col2 = col2.reshape(NS, NSB, SCH, CHUNK)
    row2 = row2.reshape(NS, NSB, SCH, CHUNK)
    a0 = a0.reshape(NS, NSB, SCH * CHUNK)
    a1 = a1.reshape(NS, NSB, SCH * CHUNK)
    a2 = a2.reshape(NS, NSB, SCH * CHUNK)
    a3 = a3.reshape(NS, NSB, SCH * CHUNK)
    xh = x.reshape(N, NC, DH).transpose(1, 0, 2)
    xp = jnp.zeros((NC, NP, DH), jnp.float32).at[:, :N].set(xh)
    x_pack = xp.reshape(NC, NPH, D)
    zeros = jnp.zeros((RPH, D), jnp.float32)
    partials = _sc_aggregate(x_pack, col2, row2, a0, a1, a2, a3, zeros)
    pr = partials.reshape(NC, NP, DH)
    return _tc_matmul(pr, weight)[:N]


def kernel(x, edge_index, adj_values, weight):
    return _impl(x, edge_index, adj_values, weight)


# speedup vs baseline: 1.5250x; 1.0300x over previous
import jax
import jax.numpy as jnp
from jax import lax
from jax.experimental import pallas as pl
from jax.experimental.pallas import tpu as pltpu
from jax.experimental.pallas import tpu_sc as plsc

N = 10000
NP = 10240
NPH = NP // 2
D = 128
DH = D // 2
E = 320000
NC = 2
NS = 16
CHUNK = 64
SCH = 32
NSB = 10
EPT = CHUNK * SCH * NSB
E_PAD = NS * EPT
RPH = NPH // NS

_LANES = 16


def _sc_body(x_hbm, col_hbm, row_hbm, a0_hbm, a1_hbm, a2_hbm, a3_hbm,
             zero_hbm, out_hbm, colv, rowv, a0v, a1v, a2v, a3v,
             buf_a, buf_b, xs, acc, ga, gb, sa, sb_sem):
    c = lax.axis_index("c")
    s = lax.axis_index("s")

    pltpu.sync_copy(x_hbm.at[c, pl.ds(s * RPH, RPH)],
                    xs.at[pl.ds(s * RPH, RPH)])
    pltpu.sync_copy(zero_hbm, acc.at[pl.ds(s * RPH, RPH)])
    plsc.subcore_barrier()

    def gather_start(k, buf, sem):
        pltpu.async_copy(xs.at[colv.at[k]], buf, sem)

    def gather_wait(buf, sem):
        pltpu.make_async_copy(xs.at[colv.at[0]], buf, sem).wait()

    def scat_start(k, buf, sem):
        pltpu.async_copy(buf, acc.at[rowv.at[k]], sem, add=True)

    def scat_wait(buf, sem):
        pltpu.make_async_copy(buf, acc.at[rowv.at[0]], sem).wait()

    def scale_chunk(k, rows):
        base = k * CHUNK
        for g in range(CHUNK // _LANES):
            o = base + g * _LANES
            a0g = a0v[pl.ds(o, _LANES)]
            a1g = a1v[pl.ds(o, _LANES)]
            a2g = a2v[pl.ds(o, _LANES)]
            a3g = a3v[pl.ds(o, _LANES)]
            for j in range(_LANES):
                e = g * _LANES + j
                s0 = jnp.full((_LANES,), a0g[j], jnp.float32)
                s1 = jnp.full((_LANES,), a1g[j], jnp.float32)
                s2 = jnp.full((_LANES,), a2g[j], jnp.float32)
                s3 = jnp.full((_LANES,), a3g[j], jnp.float32)
                for v in range(DH // _LANES):
                    sl_lo = pl.ds(v * _LANES, _LANES)
                    sl_hi = pl.ds(DH + v * _LANES, _LANES)
                    lo = rows[e, sl_lo]
                    hi = rows[e, sl_hi]
                    rows[e, sl_lo] = lo * s0 + hi * s1
                    rows[e, sl_hi] = lo * s2 + hi * s3

    def sb_body(sb, carry):
        pltpu.sync_copy(col_hbm.at[s, sb], colv)
        pltpu.sync_copy(row_hbm.at[s, sb], rowv)
        pltpu.sync_copy(a0_hbm.at[s, sb], a0v)
        pltpu.sync_copy(a1_hbm.at[s, sb], a1v)
        pltpu.sync_copy(a2_hbm.at[s, sb], a2v)
        pltpu.sync_copy(a3_hbm.at[s, sb], a3v)

        @pl.when(sb > 0)
        def _():
            scat_wait(buf_a, sa)

        gather_start(0, buf_a, ga)

        def pair_body(j, carry2):
            k0 = 2 * j
            k1 = k0 + 1
            gather_wait(buf_a, ga)
            scale_chunk(k0, buf_a)
            scat_start(k0, buf_a, sa)

            not_first = jnp.logical_or(sb > 0, j > 0)

            @pl.when(not_first)
            def _():
                scat_wait(buf_b, sb_sem)
                gather_start(k1, buf_b, gb)

            @pl.when(jnp.logical_not(not_first))
            def _():
                gather_start(k1, buf_b, gb)

            gather_wait(buf_b, gb)
            scale_chunk(k1, buf_b)
            scat_start(k1, buf_b, sb_sem)

            @pl.when(k0 + 2 < SCH)
            def _():
                scat_wait(buf_a, sa)
                gather_start(k0 + 2, buf_a, ga)

            return carry2

        lax.fori_loop(0, SCH // 2, pair_body, 0)
        return carry

    lax.fori_loop(0, NSB, sb_body, 0)
    scat_wait(buf_a, sa)
    scat_wait(buf_b, sb_sem)
    plsc.subcore_barrier()
    pltpu.sync_copy(acc.at[pl.ds(s * RPH, RPH)],
                    out_hbm.at[c, pl.ds(s * RPH, RPH)])


_sc_aggregate = pl.kernel(
    _sc_body,
    out_type=jax.ShapeDtypeStruct((NC, NPH, D), jnp.float32),
    mesh=plsc.VectorSubcoreMesh(
        core_axis_name="c", subcore_axis_name="s",
        num_cores=NC, num_subcores=NS),
    scratch_types=[
        pltpu.VMEM((SCH, CHUNK), jnp.int32),
        pltpu.VMEM((SCH, CHUNK), jnp.int32),
        pltpu.VMEM((SCH * CHUNK,), jnp.float32),
        pltpu.VMEM((SCH * CHUNK,), jnp.float32),
        pltpu.VMEM((SCH * CHUNK,), jnp.float32),
        pltpu.VMEM((SCH * CHUNK,), jnp.float32),
        pltpu.VMEM((CHUNK, D), jnp.float32),
        pltpu.VMEM((CHUNK, D), jnp.float32),
        pltpu.VMEM_SHARED((NPH, D), jnp.float32),
        pltpu.VMEM_SHARED((NPH, D), jnp.float32),
        pltpu.SemaphoreType.DMA,
        pltpu.SemaphoreType.DMA,
        pltpu.SemaphoreType.DMA,
        pltpu.SemaphoreType.DMA,
    ],
)

_BM = 1024


def _tc_body(p_ref, w_ref, o_ref):
    o_ref[...] = (
        jnp.dot(p_ref[0], w_ref[pl.ds(0, DH), :],
                preferred_element_type=jnp.float32)
        + jnp.dot(p_ref[1], w_ref[pl.ds(DH, DH), :],
                  preferred_element_type=jnp.float32))


def _tc_matmul(partials, weight):
    return pl.pallas_call(
        _tc_body,
        grid=(NP // _BM,),
        in_specs=[
            pl.BlockSpec((NC, _BM, DH), lambda i: (0, i, 0)),
            pl.BlockSpec((D, D), lambda i: (0, 0)),
        ],
        out_specs=pl.BlockSpec((_BM, D), lambda i: (i, 0)),
        out_shape=jax.ShapeDtypeStruct((NP, D), jnp.float32),
    )(partials, weight)


@jax.jit
def _impl(x, edge_index, adj_values, weight):
    row = edge_index[0]
    col = edge_index[1]
    col2 = jnp.zeros((E_PAD,), jnp.int32).at[:E].set(col >> 1)
    row2 = jnp.zeros((E_PAD,), jnp.int32).at[:E].set(row >> 1)
    cp = (col & 1).astype(jnp.float32)
    rp = (row & 1).astype(jnp.float32)
    a0 = jnp.zeros((E_PAD,), jnp.float32).at[:E].set(
        adj_values * (1 - cp) * (1 - rp))
    a1 = jnp.zeros((E_PAD,), jnp.float32).at[:E].set(
        adj_values * cp * (1 - rp))
    a2 = jnp.zeros((E_PAD,), jnp.float32).at[:E].set(
        adj_values * (1 - cp) * rp)
    a3 = jnp.zeros((E_PAD,), jnp.float32).at[:E].set(
        adj_values * cp * rp)
    col2 = col2.reshape(NS, NSB, SCH, CHUNK)
    row2 = row2.reshape(NS, NSB, SCH, CHUNK)
    a0 = a0.reshape(NS, NSB, SCH * CHUNK)
    a1 = a1.reshape(NS, NSB, SCH * CHUNK)
    a2 = a2.reshape(NS, NSB, SCH * CHUNK)
    a3 = a3.reshape(NS, NSB, SCH * CHUNK)
    xh = x.reshape(N, NC, DH).transpose(1, 0, 2)
    xp = jnp.zeros((NC, NP, DH), jnp.float32).at[:, :N].set(xh)
    x_pack = xp.reshape(NC, NPH, D)
    zeros = jnp.zeros((RPH, D), jnp.float32)
    partials = _sc_aggregate(x_pack, col2, row2, a0, a1, a2, a3, zeros)
    pr = partials.reshape(NC, NP, DH)
    return _tc_matmul(pr, weight)[:N]


def kernel(x, edge_index, adj_values, weight):
    return _impl(x, edge_index, adj_values, weight)
